# baseline (speedup 1.0000x reference)
import functools

import jax
import jax.numpy as jnp
from jax import lax
from jax.experimental import pallas as pl
from jax.experimental.pallas import tpu as pltpu

_THRESHOLD = 0.1
_ONE_BITS = 0x3F800001


def _sigmoid(z):
    return 1.0 / (1.0 + jnp.exp(-z))


def _body(L, HW, K, CB, keep_ref, ego_ref, cav_ref, x_ref, of_ref, or_ref,
          oa_ref, mm, mr, ma):
    b = pl.program_id(0)
    i = pl.program_id(1)
    cb = pl.program_id(2)

    @pl.when(cb == 0)
    def _compute_masks():
        @pl.when(i == 0)
        def _ones():
            ones = jnp.ones((8, HW), jnp.float32)
            mm[:] = ones
            mr[:] = ones
            ma[:] = ones

        @pl.when(i != 0)
        def _topk_masks():
            ego = ego_ref[0, 0]
            cav = cav_ref[0, 0]
            r = jnp.max(_sigmoid(cav - ego), axis=0, keepdims=True)
            a = jnp.max(_sigmoid(cav + ego), axis=0, keepdims=True)
            bits = lax.bitcast_convert_type(
                jnp.concatenate([r, a], axis=0), jnp.int32)

            def step(_, lohi):
                lo, hi = lohi
                mid = lo + (hi - lo) // 2
                cnt = jnp.sum((bits >= mid).astype(jnp.int32), axis=1,
                              keepdims=True)
                pred = cnt >= K
                return (jnp.where(pred, mid, lo), jnp.where(pred, hi, mid))

            init = (jnp.zeros((2, 1), jnp.int32),
                    jnp.full((2, 1), _ONE_BITS, jnp.int32))
            lo, _ = lax.fori_loop(0, 31, step, init)

            kf = jnp.where(keep_ref[b * L + i] != 0, jnp.float32(1.0),
                           jnp.float32(0.0))
            fra = (bits >= lo).astype(jnp.float32) * kf
            fr = fra[0:1]
            fa = fra[1:2]
            mr[:] = jnp.broadcast_to(fr, (8, HW))
            ma[:] = jnp.broadcast_to(fa, (8, HW))
            mm[:] = jnp.broadcast_to(jnp.maximum(fr, fa), (8, HW))

    mmv = mm[:]
    mrv = mr[:]
    mav = ma[:]
    for j in range(CB // 8):
        sl = pl.ds(j * 8, 8)
        xs = x_ref[0, 0, sl]
        of_ref[0, 0, sl] = xs * mmv
        or_ref[0, 0, sl] = xs * mrv
        oa_ref[0, 0, sl] = xs * mav


def kernel(x, psm, mask, flag):
    B, L, C, H, W = x.shape
    P = psm.shape[2]
    HW = H * W
    K = max(1, int(HW * _THRESHOLD))
    CB = 64 if C % 64 == 0 else C
    NCB = C // CB

    x4 = x.reshape(B, L, C, HW)
    psm4 = psm.reshape(B, L, P, HW)
    keep = ((mask * jnp.asarray(flag, mask.dtype)) != 0).astype(
        jnp.int32).reshape(-1)

    psm_spec_ego = pl.BlockSpec((1, 1, P, HW), lambda b, i, cb, *_: (b, 0, 0, 0))
    psm_spec_cav = pl.BlockSpec((1, 1, P, HW), lambda b, i, cb, *_: (b, i, 0, 0))
    x_spec = pl.BlockSpec((1, 1, CB, HW), lambda b, i, cb, *_: (b, i, cb, 0))

    grid_spec = pltpu.PrefetchScalarGridSpec(
        num_scalar_prefetch=1,
        grid=(B, L, NCB),
        in_specs=[psm_spec_ego, psm_spec_cav, x_spec],
        out_specs=[x_spec, x_spec, x_spec],
        scratch_shapes=[pltpu.VMEM((8, HW), jnp.float32)] * 3,
    )

    outs = pl.pallas_call(
        functools.partial(_body, L, HW, K, CB),
        grid_spec=grid_spec,
        out_shape=[jax.ShapeDtypeStruct((B, L, C, HW), jnp.float32)] * 3,
        compiler_params=pltpu.CompilerParams(
            dimension_semantics=("arbitrary", "arbitrary", "arbitrary")),
    )(keep, psm4, psm4, x4)

    return tuple(o.reshape(B, L, C, H, W) for o in outs)
